# Initial kernel scaffold; baseline (speedup 1.0000x reference)
import jax
import jax.numpy as jnp
from jax.experimental import pallas as pl


def kernel(Xi, Xv, fo_lin_w, fo_lin_b, fo_emb, so_lin_w, so_lin_b, so_emb, lin1_w, lin1_b, bn1_g, bn1_b, lin2_w, lin2_b, bn2_g, bn2_b, bias):
    raise NotImplementedError("write your pallas kernel here")


# speedup vs baseline: 20.7848x; 20.7848x over previous
import functools
import math

import jax
import jax.numpy as jnp
from jax import lax
from jax.experimental import pallas as pl
from jax.experimental.pallas import tpu as pltpu
from jax.experimental.pallas import tpu_sc as plsc

B = 4096
F = 39
ND = 13
NS = 26
V = 1000
EMB = 128
H1 = 1024
H2 = 512
DIN = F * EMB
C_BN = float(1.0 / math.sqrt(1.0 + 1e-5))
_HI = lax.Precision.HIGHEST

SC_CORES = 2
SC_SUBCORES = 16
NW = SC_CORES * SC_SUBCORES
NROWS = B * NS
CH = 128
NHALF = 2
BH = B // NHALF
RPWH = BH * NS // NW
NCH_H = RPWH // CH
BBM = BH // NW


NBUF = 5
RFW = 16
LPC = CH // 16
RTAB_R = (NS * V + 127) // 128
EMBW = EMB // 2


def _sc_gather(soT, rfo1d, jdx3, nchunk):
    mesh = plsc.VectorSubcoreMesh(core_axis_name="c", subcore_axis_name="s")
    rpw = nchunk * CH
    nrows = NW * rpw

    @functools.partial(
        pl.kernel,
        mesh=mesh,
        out_type=(
            jax.ShapeDtypeStruct((nrows, EMB), jnp.float32),
            jax.ShapeDtypeStruct((nrows,), jnp.float32),
        ),
        scratch_types=(
            [pltpu.VMEM((nchunk, CH), jnp.int32)]
            + [pltpu.VMEM((CH, EMB), jnp.float32) for _ in range(NBUF)]
            + [pltpu.VMEM((CH,), jnp.float32) for _ in range(NBUF)]
            + [pltpu.SemaphoreType.DMA for _ in range(4 * NBUF)]
        ),
    )
    def k(so_hbm, rfo_hbm, jdx_hbm, oso, orfo, idx_v, *bufs):
        sbufs = bufs[:NBUF]
        rbufs = bufs[NBUF:2 * NBUF]
        gs = bufs[2 * NBUF:3 * NBUF]
        gr = bufs[3 * NBUF:4 * NBUF]
        ss = bufs[4 * NBUF:5 * NBUF]
        sr = bufs[5 * NBUF:6 * NBUF]
        wid = lax.axis_index("s") * SC_CORES + lax.axis_index("c")
        base = wid * rpw
        pltpu.sync_copy(jdx_hbm.at[wid], idx_v)

        gso_h = [None] * NBUF
        grf_h = [None] * NBUF
        sso_h = [None] * NBUF
        srf_h = [None] * NBUF

        def start_gather(i):
            b = i % NBUF
            gso_h[b] = pltpu.async_copy(so_hbm.at[idx_v.at[i]], sbufs[b], gs[b])
            grf_h[b] = pltpu.async_copy(rfo_hbm.at[idx_v.at[i]], rbufs[b], gr[b])

        def drain_and_scatter(j):
            b = j % NBUF
            gso_h[b].wait()
            grf_h[b].wait()
            off = base + j * CH
            sso_h[b] = pltpu.async_copy(sbufs[b], oso.at[pl.ds(off, CH)], ss[b])
            srf_h[b] = pltpu.async_copy(rbufs[b], orfo.at[pl.ds(off, CH)], sr[b])

        for i in range(min(NBUF - 1, nchunk)):
            start_gather(i)
        for i in range(NBUF - 1, nchunk):
            b = i % NBUF
            if sso_h[b] is not None:
                sso_h[b].wait()
                srf_h[b].wait()
                sso_h[b] = None
                srf_h[b] = None
            start_gather(i)
            drain_and_scatter(i - (NBUF - 1))
        for j in range(max(0, nchunk - (NBUF - 1)), nchunk):
            drain_and_scatter(j)
        for b in range(NBUF):
            if sso_h[b] is not None:
                sso_h[b].wait()
                srf_h[b].wait()

    return k(soT, rfo1d, jdx3)


def _rowsum_body(tab, out):
    out[...] = jnp.dot(tab[...], jnp.ones((EMB, RFW), jnp.float32))


def _collapse_body(l2w, g2, g1, b1, bb1, b2, bb2, u_out, k_out):
    c2 = g2[...] * C_BN
    w2v = lax.dot_general(c2, l2w[...], (((1,), (1,)), ((), ())),
                          precision=_HI)
    u = w2v * g1[...] * C_BN
    kconst = (jnp.sum(b1[...] * u, axis=1, keepdims=True)
              + jnp.sum(bb1[...] * w2v, axis=1, keepdims=True)
              + jnp.sum(b2[...] * c2, axis=1, keepdims=True)
              + jnp.sum(bb2[...], axis=1, keepdims=True))
    u_out[...] = u
    k_out[...] = kconst


def _w1v_body(u, w1, out):
    out[...] = lax.dot_general(u[...], w1[...], (((1,), (1,)), ((), ())))


def _main_body(gso, grf, xd, vd, vs, sw, sb, fw, fb, wv, kc, bias, out):
    xd_ = xd[...]
    vd_ = vd[...]
    vs_ = vs[...]
    vx = vd_ * xd_
    sw_ = sw[...]
    sb_ = sb[...]
    wv_ = wv[...]
    wv_d = wv_[:ND]

    S = (jnp.dot(vx, sw_, precision=_HI)
         + jnp.dot(vd_, sb_, precision=_HI))
    s1 = (jnp.dot(vx, fw[...], precision=_HI)
          + jnp.dot(vd_, fb[...], precision=_HI))

    sww = jnp.sum(sw_ * sw_, axis=1, keepdims=True)
    swb = jnp.sum(sw_ * sb_, axis=1, keepdims=True)
    sbb = jnp.sum(sb_ * sb_, axis=1, keepdims=True)
    v2 = vd_ * vd_
    sqd = (jnp.dot(v2 * xd_ * xd_, sww, precision=_HI)
           + 2.0 * jnp.dot(v2 * xd_, swb, precision=_HI)
           + jnp.dot(v2, sbb, precision=_HI))
    swv = jnp.sum(sw_ * wv_d, axis=1, keepdims=True)
    sbv = jnp.sum(sb_ * wv_d, axis=1, keepdims=True)
    dd = (jnp.dot(vx, swv, precision=_HI)
          + jnp.dot(vd_, sbv, precision=_HI))

    sq128 = jnp.zeros_like(S)
    d128 = jnp.zeros_like(S)
    s1_sc = jnp.sum(vs_ * grf[...], axis=1, keepdims=True)
    for j in range(NS):
        vj = vs_[:, j:j + 1]
        t = vj * gso[0, j * BBM:(j + 1) * BBM, :]
        S = S + t
        sq128 = sq128 + t * t
        d128 = d128 + t * wv_[ND + j:ND + j + 1, :]

    tot = (s1_sc + jnp.sum(s1, axis=1, keepdims=True)
           + 0.5 * (jnp.sum(S * S, axis=1, keepdims=True)
                    - jnp.sum(sq128, axis=1, keepdims=True))
           - 0.5 * sqd
           + jnp.sum(d128, axis=1, keepdims=True) + dd
           + kc[...]
           + bias[...])
    out[...] = tot


def kernel(Xi, Xv, fo_lin_w, fo_lin_b, fo_emb, so_lin_w, so_lin_b, so_emb,
           lin1_w, lin1_b, bn1_g, bn1_b, lin2_w, lin2_b, bn2_g, bn2_b, bias):
    idx = Xi[:, ND:, 0]
    jdxh = []
    for h in range(NHALF):
        ih = idx[h * BH:(h + 1) * BH]
        j3 = (ih.reshape(NW, BBM, NS).transpose(0, 2, 1)
              + (jnp.arange(NS, dtype=jnp.int32) * V)[None, :, None])
        jdxh.append(j3.reshape(NW, NCH_H, CH))
    xd = Xi[:, :ND, 0].astype(jnp.float32)
    vd = Xv[:, :ND]
    vs = Xv[:, ND:]
    soT = so_emb.reshape(NS * V, EMB)
    foT = fo_emb.reshape(NS * V, EMB)
    g1r = bn1_g.reshape(1, H1)
    g2r = bn2_g.reshape(1, H2)
    b1r = lin1_b.reshape(1, H1)
    bb1r = bn1_b.reshape(1, H1)
    b2r = lin2_b.reshape(1, H2)
    bb2r = bn2_b.reshape(1, H2)
    bias2 = bias.reshape(B, 1)

    rfoT = pl.pallas_call(
        _rowsum_body,
        grid=(NS,),
        in_specs=[pl.BlockSpec((V, EMB), lambda i: (i, 0))],
        out_specs=pl.BlockSpec((V, RFW), lambda i: (i, 0)),
        out_shape=jax.ShapeDtypeStruct((NS * V, RFW), jnp.float32),
    )(foT)
    rfo1d = rfoT[:, 0]

    gso3h = []
    grf2h = []
    for h in range(NHALF):
        gso_flat, grf_flat = _sc_gather(soT, rfo1d, jdxh[h], NCH_H)
        gso3h.append(gso_flat.reshape(NW, NS * BBM, EMB))
        grf2h.append(grf_flat.reshape(NW, NCH_H, 2, BBM)
                     .transpose(0, 3, 1, 2).reshape(BH, NS))

    u_row, kc = pl.pallas_call(
        _collapse_body,
        out_shape=(jax.ShapeDtypeStruct((1, H1), jnp.float32),
                   jax.ShapeDtypeStruct((1, 1), jnp.float32)),
    )(lin2_w, g2r, g1r, b1r, bb1r, b2r, bb2r)

    w1v_row = pl.pallas_call(
        _w1v_body,
        grid=(F,),
        in_specs=[
            pl.BlockSpec((1, H1), lambda i: (0, 0)),
            pl.BlockSpec((EMB, H1), lambda i: (i, 0)),
        ],
        out_specs=pl.BlockSpec((1, EMB), lambda i: (0, i)),
        out_shape=jax.ShapeDtypeStruct((1, DIN), jnp.float32),
    )(u_row, lin1_w)
    wv = w1v_row.reshape(F, EMB)

    bb = BBM
    grid = BH // bb
    totals = []
    for h in range(NHALF):
        sl = slice(h * BH, (h + 1) * BH)
        total_h = pl.pallas_call(
            _main_body,
            grid=(grid,),
            in_specs=[
                pl.BlockSpec((1, NS * BBM, EMB), lambda i: (i, 0, 0)),
                pl.BlockSpec((bb, NS), lambda i: (i, 0)),
                pl.BlockSpec((bb, ND), lambda i: (i, 0)),
                pl.BlockSpec((bb, ND), lambda i: (i, 0)),
                pl.BlockSpec((bb, NS), lambda i: (i, 0)),
                pl.BlockSpec((ND, EMB), lambda i: (0, 0)),
                pl.BlockSpec((ND, EMB), lambda i: (0, 0)),
                pl.BlockSpec((ND, EMB), lambda i: (0, 0)),
                pl.BlockSpec((ND, EMB), lambda i: (0, 0)),
                pl.BlockSpec((F, EMB), lambda i: (0, 0)),
                pl.BlockSpec((1, 1), lambda i: (0, 0)),
                pl.BlockSpec((bb, 1), lambda i: (i, 0)),
            ],
            out_specs=pl.BlockSpec((bb, 1), lambda i: (i, 0)),
            out_shape=jax.ShapeDtypeStruct((BH, 1), jnp.float32),
        )(gso3h[h], grf2h[h], xd[sl], vd[sl], vs[sl], so_lin_w, so_lin_b,
          fo_lin_w, fo_lin_b, wv, kc, bias2[sl])
        totals.append(total_h)

    return jnp.concatenate(totals, axis=0).reshape(B)


# speedup vs baseline: 24.2116x; 1.1649x over previous
import functools
import math

import jax
import jax.numpy as jnp
from jax import lax
from jax.experimental import pallas as pl
from jax.experimental.pallas import tpu as pltpu
from jax.experimental.pallas import tpu_sc as plsc

B = 4096
F = 39
ND = 13
NS = 26
V = 1000
EMB = 128
H1 = 1024
H2 = 512
DIN = F * EMB
C_BN = float(1.0 / math.sqrt(1.0 + 1e-5))
_HI = lax.Precision.HIGHEST

SC_CORES = 2
SC_SUBCORES = 16
NW = SC_CORES * SC_SUBCORES
NROWS = B * NS
CH = 128
NHALF = 1
BH = B // NHALF
RPWH = BH * NS // NW
NCH_H = RPWH // CH
BBM = BH // NW


NBUF = 5
RFW = 16
LPC = CH // 16
RTAB_R = (NS * V + 127) // 128
EMBW = EMB // 2


def _sc_gather(soT, rfo1d, jdx3, nchunk):
    mesh = plsc.VectorSubcoreMesh(core_axis_name="c", subcore_axis_name="s")
    rpw = nchunk * CH
    nrows = NW * rpw

    @functools.partial(
        pl.kernel,
        mesh=mesh,
        out_type=(
            jax.ShapeDtypeStruct((nrows, EMB), jnp.float32),
            jax.ShapeDtypeStruct((nrows,), jnp.float32),
        ),
        scratch_types=(
            [pltpu.VMEM((nchunk, CH), jnp.int32)]
            + [pltpu.VMEM((CH, EMB), jnp.float32) for _ in range(NBUF)]
            + [pltpu.VMEM((CH,), jnp.float32) for _ in range(NBUF)]
            + [pltpu.SemaphoreType.DMA for _ in range(4 * NBUF)]
        ),
    )
    def k(so_hbm, rfo_hbm, jdx_hbm, oso, orfo, idx_v, *bufs):
        sbufs = bufs[:NBUF]
        rbufs = bufs[NBUF:2 * NBUF]
        gs = bufs[2 * NBUF:3 * NBUF]
        gr = bufs[3 * NBUF:4 * NBUF]
        ss = bufs[4 * NBUF:5 * NBUF]
        sr = bufs[5 * NBUF:6 * NBUF]
        wid = lax.axis_index("s") * SC_CORES + lax.axis_index("c")
        base = wid * rpw
        pltpu.sync_copy(jdx_hbm.at[wid], idx_v)

        gso_h = [None] * NBUF
        grf_h = [None] * NBUF
        sso_h = [None] * NBUF
        srf_h = [None] * NBUF

        def start_gather(i):
            b = i % NBUF
            gso_h[b] = pltpu.async_copy(so_hbm.at[idx_v.at[i]], sbufs[b], gs[b])
            grf_h[b] = pltpu.async_copy(rfo_hbm.at[idx_v.at[i]], rbufs[b], gr[b])

        def drain_and_scatter(j):
            b = j % NBUF
            gso_h[b].wait()
            grf_h[b].wait()
            off = base + j * CH
            sso_h[b] = pltpu.async_copy(sbufs[b], oso.at[pl.ds(off, CH)], ss[b])
            srf_h[b] = pltpu.async_copy(rbufs[b], orfo.at[pl.ds(off, CH)], sr[b])

        for i in range(min(NBUF - 1, nchunk)):
            start_gather(i)
        for i in range(NBUF - 1, nchunk):
            b = i % NBUF
            if sso_h[b] is not None:
                sso_h[b].wait()
                srf_h[b].wait()
                sso_h[b] = None
                srf_h[b] = None
            start_gather(i)
            drain_and_scatter(i - (NBUF - 1))
        for j in range(max(0, nchunk - (NBUF - 1)), nchunk):
            drain_and_scatter(j)
        for b in range(NBUF):
            if sso_h[b] is not None:
                sso_h[b].wait()
                srf_h[b].wait()

    return k(soT, rfo1d, jdx3)


def _rowsum_body(tab, out):
    out[...] = jnp.dot(tab[...], jnp.ones((EMB, RFW), jnp.float32))


def _collapse_body(l2w, g2, g1, b1, bb1, b2, bb2, u_out, k_out):
    c2 = g2[...] * C_BN
    w2v = lax.dot_general(c2, l2w[...], (((1,), (1,)), ((), ())),
                          precision=_HI)
    u = w2v * g1[...] * C_BN
    kconst = (jnp.sum(b1[...] * u, axis=1, keepdims=True)
              + jnp.sum(bb1[...] * w2v, axis=1, keepdims=True)
              + jnp.sum(b2[...] * c2, axis=1, keepdims=True)
              + jnp.sum(bb2[...], axis=1, keepdims=True))
    u_out[...] = u
    k_out[...] = kconst


def _w1v_body(u, w1, out):
    out[...] = lax.dot_general(u[...], w1[...], (((1,), (1,)), ((), ())))


def _main_body(gso, grf, xd, vd, vs, sw, sb, fw, fb, wv, kc, bias, out):
    xd_ = xd[...]
    vd_ = vd[...]
    vs_ = vs[...]
    vx = vd_ * xd_
    sw_ = sw[...]
    sb_ = sb[...]
    wv_ = wv[...]
    wv_d = wv_[:ND]

    S = (jnp.dot(vx, sw_, precision=_HI)
         + jnp.dot(vd_, sb_, precision=_HI))
    s1 = (jnp.dot(vx, fw[...], precision=_HI)
          + jnp.dot(vd_, fb[...], precision=_HI))

    sww = jnp.sum(sw_ * sw_, axis=1, keepdims=True)
    swb = jnp.sum(sw_ * sb_, axis=1, keepdims=True)
    sbb = jnp.sum(sb_ * sb_, axis=1, keepdims=True)
    v2 = vd_ * vd_
    sqd = (jnp.dot(v2 * xd_ * xd_, sww, precision=_HI)
           + 2.0 * jnp.dot(v2 * xd_, swb, precision=_HI)
           + jnp.dot(v2, sbb, precision=_HI))
    swv = jnp.sum(sw_ * wv_d, axis=1, keepdims=True)
    sbv = jnp.sum(sb_ * wv_d, axis=1, keepdims=True)
    dd = (jnp.dot(vx, swv, precision=_HI)
          + jnp.dot(vd_, sbv, precision=_HI))

    sq128 = jnp.zeros_like(S)
    d128 = jnp.zeros_like(S)
    s1_sc = jnp.sum(vs_ * grf[...], axis=1, keepdims=True)
    for j in range(NS):
        vj = vs_[:, j:j + 1]
        t = vj * gso[0, j * BBM:(j + 1) * BBM, :]
        S = S + t
        sq128 = sq128 + t * t
        d128 = d128 + t * wv_[ND + j:ND + j + 1, :]

    tot = (s1_sc + jnp.sum(s1, axis=1, keepdims=True)
           + 0.5 * (jnp.sum(S * S, axis=1, keepdims=True)
                    - jnp.sum(sq128, axis=1, keepdims=True))
           - 0.5 * sqd
           + jnp.sum(d128, axis=1, keepdims=True) + dd
           + kc[...]
           + bias[...])
    out[...] = tot


def kernel(Xi, Xv, fo_lin_w, fo_lin_b, fo_emb, so_lin_w, so_lin_b, so_emb,
           lin1_w, lin1_b, bn1_g, bn1_b, lin2_w, lin2_b, bn2_g, bn2_b, bias):
    idx = Xi[:, ND:, 0]
    jdxh = []
    for h in range(NHALF):
        ih = idx[h * BH:(h + 1) * BH]
        j3 = (ih.reshape(NW, BBM, NS).transpose(0, 2, 1)
              + (jnp.arange(NS, dtype=jnp.int32) * V)[None, :, None])
        jdxh.append(j3.reshape(NW, NCH_H, CH))
    xd = Xi[:, :ND, 0].astype(jnp.float32)
    vd = Xv[:, :ND]
    vs = Xv[:, ND:]
    soT = so_emb.reshape(NS * V, EMB)
    foT = fo_emb.reshape(NS * V, EMB)
    g1r = bn1_g.reshape(1, H1)
    g2r = bn2_g.reshape(1, H2)
    b1r = lin1_b.reshape(1, H1)
    bb1r = bn1_b.reshape(1, H1)
    b2r = lin2_b.reshape(1, H2)
    bb2r = bn2_b.reshape(1, H2)
    bias2 = bias.reshape(B, 1)

    rfoT = pl.pallas_call(
        _rowsum_body,
        grid=(NS,),
        in_specs=[pl.BlockSpec((V, EMB), lambda i: (i, 0))],
        out_specs=pl.BlockSpec((V, RFW), lambda i: (i, 0)),
        out_shape=jax.ShapeDtypeStruct((NS * V, RFW), jnp.float32),
    )(foT)
    rfo1d = rfoT[:, 0]

    gso3h = []
    grf2h = []
    for h in range(NHALF):
        gso_flat, grf_flat = _sc_gather(soT, rfo1d, jdxh[h], NCH_H)
        gso3h.append(gso_flat.reshape(NW, NS * BBM, EMB))
        grf2h.append(grf_flat.reshape(NW, NS, BBM)
                     .transpose(0, 2, 1).reshape(BH, NS))

    u_row, kc = pl.pallas_call(
        _collapse_body,
        out_shape=(jax.ShapeDtypeStruct((1, H1), jnp.float32),
                   jax.ShapeDtypeStruct((1, 1), jnp.float32)),
    )(lin2_w, g2r, g1r, b1r, bb1r, b2r, bb2r)

    w1v_row = pl.pallas_call(
        _w1v_body,
        grid=(F,),
        in_specs=[
            pl.BlockSpec((1, H1), lambda i: (0, 0)),
            pl.BlockSpec((EMB, H1), lambda i: (i, 0)),
        ],
        out_specs=pl.BlockSpec((1, EMB), lambda i: (0, i)),
        out_shape=jax.ShapeDtypeStruct((1, DIN), jnp.float32),
    )(u_row, lin1_w)
    wv = w1v_row.reshape(F, EMB)

    bb = BBM
    grid = BH // bb
    totals = []
    for h in range(NHALF):
        sl = slice(h * BH, (h + 1) * BH)
        total_h = pl.pallas_call(
            _main_body,
            grid=(grid,),
            in_specs=[
                pl.BlockSpec((1, NS * BBM, EMB), lambda i: (i, 0, 0)),
                pl.BlockSpec((bb, NS), lambda i: (i, 0)),
                pl.BlockSpec((bb, ND), lambda i: (i, 0)),
                pl.BlockSpec((bb, ND), lambda i: (i, 0)),
                pl.BlockSpec((bb, NS), lambda i: (i, 0)),
                pl.BlockSpec((ND, EMB), lambda i: (0, 0)),
                pl.BlockSpec((ND, EMB), lambda i: (0, 0)),
                pl.BlockSpec((ND, EMB), lambda i: (0, 0)),
                pl.BlockSpec((ND, EMB), lambda i: (0, 0)),
                pl.BlockSpec((F, EMB), lambda i: (0, 0)),
                pl.BlockSpec((1, 1), lambda i: (0, 0)),
                pl.BlockSpec((bb, 1), lambda i: (i, 0)),
            ],
            out_specs=pl.BlockSpec((bb, 1), lambda i: (i, 0)),
            out_shape=jax.ShapeDtypeStruct((BH, 1), jnp.float32),
        )(gso3h[h], grf2h[h], xd[sl], vd[sl], vs[sl], so_lin_w, so_lin_b,
          fo_lin_w, fo_lin_b, wv, kc, bias2[sl])
        totals.append(total_h)

    return jnp.concatenate(totals, axis=0).reshape(B)
